# Initial kernel scaffold; baseline (speedup 1.0000x reference)
import jax
import jax.numpy as jnp
from jax.experimental import pallas as pl


def kernel(features, edge_index, edge_weight, W_self0, W_neigh0, b0, W_self1, W_neigh1, b1):
    raise NotImplementedError("write your pallas kernel here")


# speedup vs baseline: 5.3840x; 5.3840x over previous
import functools

import jax
import jax.numpy as jnp
from jax import lax
from jax.experimental import pallas as pl
from jax.experimental.pallas import tpu as pltpu
from jax.experimental.pallas import tpu_sc as plsc

N = 10000
E = 320000
D_IN = 128
D_HID = 128
D_OUT = 64

NC = 2
NS = 16
NW = NC * NS
PERW = E // NW
C = 80
NCH = PERW // C
RPT = 624
RTAIL = N - NS * RPT

_f32 = jnp.float32


def _pre_body(x_ref, wn_ref, ws_ref, b_ref, z_ref, u_ref):
    x = x_ref[...]
    z_ref[...] = jnp.dot(x, wn_ref[...], preferred_element_type=_f32)
    u_ref[...] = jnp.dot(x, ws_ref[...], preferred_element_type=_f32) + b_ref[...]


def _mid_body(u0_ref, pa_ref, pb_ref, da_ref, db_ref, wn_ref, ws_ref, b_ref,
              z_ref, u_ref):
    deg = jnp.maximum(da_ref[...][:, 0:1] + db_ref[...][:, 0:1], 1.0)
    h = jnp.maximum(u0_ref[...] + (pa_ref[...] + pb_ref[...]) / deg, 0.0)
    z_ref[...] = jnp.dot(h, wn_ref[...], preferred_element_type=_f32)
    u_ref[...] = jnp.dot(h, ws_ref[...], preferred_element_type=_f32) + b_ref[...]


def _post_body(u1_ref, pa_ref, pb_ref, da_ref, db_ref, o_ref):
    deg = jnp.maximum(da_ref[...][:, 0:1] + db_ref[...][:, 0:1], 1.0)
    o_ref[...] = u1_ref[...] + (pa_ref[...] + pb_ref[...]) / deg


_BR = 1000


def _row_spec(d):
    return pl.BlockSpec((_BR, d), lambda i: (i, 0))


def _full_spec(r, c):
    return pl.BlockSpec((r, c), lambda i: (0, 0))


def _tc_pre(x, wn, ws, b):
    return pl.pallas_call(
        _pre_body,
        grid=(N // _BR,),
        in_specs=[_row_spec(D_IN), _full_spec(D_IN, D_HID),
                  _full_spec(D_IN, D_HID), _full_spec(1, D_HID)],
        out_specs=[_row_spec(D_HID), _row_spec(D_HID)],
        out_shape=[jax.ShapeDtypeStruct((N, D_HID), _f32)] * 2,
    )(x, wn, ws, b)


def _tc_mid(u0, pa, pb, da, db, wn, ws, b):
    return pl.pallas_call(
        _mid_body,
        grid=(N // _BR,),
        in_specs=[_row_spec(D_HID), _row_spec(D_HID), _row_spec(D_HID),
                  _row_spec(16), _row_spec(16),
                  _full_spec(D_HID, D_OUT), _full_spec(D_HID, D_OUT),
                  _full_spec(1, D_OUT)],
        out_specs=[_row_spec(D_OUT), _row_spec(D_OUT)],
        out_shape=[jax.ShapeDtypeStruct((N, D_OUT), _f32)] * 2,
    )(u0, pa, pb, da, db, wn, ws, b)


def _tc_post(u1, pa, pb, da, db):
    return pl.pallas_call(
        _post_body,
        grid=(N // _BR,),
        in_specs=[_row_spec(D_OUT), _row_spec(D_OUT), _row_spec(D_OUT),
                  _row_spec(16), _row_spec(16)],
        out_specs=_row_spec(D_OUT),
        out_shape=jax.ShapeDtypeStruct((N, D_OUT), _f32),
    )(u1, pa, pb, da, db)


def _sc0_body(z0_hbm, ed_hbm, zb_hbm, zb16_hbm,
              pa_hbm, pb_hbm, dpa_hbm, dpb_hbm,
              edv, rows, onesv, acc, dacc, sem):
    c = lax.axis_index("c")
    s = lax.axis_index("s")
    w = c * NS + s

    @pl.when(s == 0)
    def _():
        pltpu.sync_copy(zb_hbm, acc)
        pltpu.sync_copy(zb16_hbm, dacc)

    one16 = jnp.ones((16,), _f32)
    for i in range(C):
        onesv[i, :] = one16
    plsc.subcore_barrier()

    def chunk(j, carry):
        pltpu.sync_copy(ed_hbm.at[w, j], edv)
        pltpu.async_copy(z0_hbm.at[edv.at[0]], rows, sem).wait()
        for e in range(C):
            ewb = plsc.bitcast(
                plsc.load_gather(edv, [jnp.full((16,), 2, jnp.int32),
                                       jnp.full((16,), e, jnp.int32)]), _f32)
            for d in range(D_HID // 16):
                sl = pl.ds(d * 16, 16)
                rows[e, sl] = rows[e, sl] * ewb
        pltpu.sync_copy(rows, acc.at[edv.at[1]], add=True)
        pltpu.sync_copy(onesv, dacc.at[edv.at[1]], add=True)
        return carry

    lax.fori_loop(0, NCH, chunk, 0)
    plsc.subcore_barrier()

    rs = s * RPT
    tb = NS * RPT

    @pl.when(c == 0)
    def _():
        pltpu.sync_copy(acc.at[pl.ds(rs, RPT)], pa_hbm.at[pl.ds(rs, RPT)])
        pltpu.sync_copy(dacc.at[pl.ds(rs, RPT)], dpa_hbm.at[pl.ds(rs, RPT)])

    @pl.when(c == 1)
    def _():
        pltpu.sync_copy(acc.at[pl.ds(rs, RPT)], pb_hbm.at[pl.ds(rs, RPT)])
        pltpu.sync_copy(dacc.at[pl.ds(rs, RPT)], dpb_hbm.at[pl.ds(rs, RPT)])

    @pl.when((s == NS - 1) & (c == 0))
    def _():
        pltpu.sync_copy(acc.at[pl.ds(tb, RTAIL)], pa_hbm.at[pl.ds(tb, RTAIL)])
        pltpu.sync_copy(dacc.at[pl.ds(tb, RTAIL)], dpa_hbm.at[pl.ds(tb, RTAIL)])

    @pl.when((s == NS - 1) & (c == 1))
    def _():
        pltpu.sync_copy(acc.at[pl.ds(tb, RTAIL)], pb_hbm.at[pl.ds(tb, RTAIL)])
        pltpu.sync_copy(dacc.at[pl.ds(tb, RTAIL)], dpb_hbm.at[pl.ds(tb, RTAIL)])


def _sc_agg0(z0, edr, zb, zb16):
    mesh = plsc.VectorSubcoreMesh(core_axis_name="c", subcore_axis_name="s")
    f = pl.kernel(
        _sc0_body,
        out_type=[jax.ShapeDtypeStruct((N, D_HID), _f32),
                  jax.ShapeDtypeStruct((N, D_HID), _f32),
                  jax.ShapeDtypeStruct((N, 16), _f32),
                  jax.ShapeDtypeStruct((N, 16), _f32)],
        mesh=mesh,
        compiler_params=pltpu.CompilerParams(needs_layout_passes=False, use_tc_tiling_on_sc=False),
        scratch_types=[
            pltpu.VMEM((3, C), jnp.int32),
            pltpu.VMEM((C, D_HID), _f32),
            pltpu.VMEM((C, 16), _f32),
            pltpu.VMEM_SHARED((N, D_HID), _f32),
            pltpu.VMEM_SHARED((N, 16), _f32),
            pltpu.SemaphoreType.DMA,
        ],
    )
    return f(z0, edr, zb, zb16)


def _sc1_body(z1_hbm, src_hbm, dst_hbm, zb_hbm,
              pa_hbm, pb_hbm,
              srcv, dstv, rows, acc, sem):
    c = lax.axis_index("c")
    s = lax.axis_index("s")
    w = c * NS + s
    pltpu.sync_copy(src_hbm.at[w], srcv)
    pltpu.sync_copy(dst_hbm.at[w], dstv)

    @pl.when(s == 0)
    def _():
        pltpu.sync_copy(zb_hbm, acc)

    plsc.subcore_barrier()

    def chunk(j, carry):
        pltpu.async_copy(z1_hbm.at[srcv.at[j]], rows, sem).wait()
        pltpu.sync_copy(rows, acc.at[dstv.at[j]], add=True)
        return carry

    lax.fori_loop(0, NCH, chunk, 0)
    plsc.subcore_barrier()

    rs = s * RPT
    tb = NS * RPT

    @pl.when(c == 0)
    def _():
        pltpu.sync_copy(acc.at[pl.ds(rs, RPT)], pa_hbm.at[pl.ds(rs, RPT)])

    @pl.when(c == 1)
    def _():
        pltpu.sync_copy(acc.at[pl.ds(rs, RPT)], pb_hbm.at[pl.ds(rs, RPT)])

    @pl.when((s == NS - 1) & (c == 0))
    def _():
        pltpu.sync_copy(acc.at[pl.ds(tb, RTAIL)], pa_hbm.at[pl.ds(tb, RTAIL)])

    @pl.when((s == NS - 1) & (c == 1))
    def _():
        pltpu.sync_copy(acc.at[pl.ds(tb, RTAIL)], pb_hbm.at[pl.ds(tb, RTAIL)])


def _sc_agg1(z1, srcr, dstr, zb):
    mesh = plsc.VectorSubcoreMesh(core_axis_name="c", subcore_axis_name="s")
    f = pl.kernel(
        _sc1_body,
        out_type=[jax.ShapeDtypeStruct((N, D_OUT), _f32),
                  jax.ShapeDtypeStruct((N, D_OUT), _f32)],
        mesh=mesh,
        compiler_params=pltpu.CompilerParams(needs_layout_passes=False, use_tc_tiling_on_sc=False),
        scratch_types=[
            pltpu.VMEM((NCH, C), jnp.int32),
            pltpu.VMEM((NCH, C), jnp.int32),
            pltpu.VMEM((C, D_OUT), _f32),
            pltpu.VMEM_SHARED((N, D_OUT), _f32),
            pltpu.SemaphoreType.DMA,
        ],
    )
    return f(z1, srcr, dstr, zb)


@jax.jit
def kernel(features, edge_index, edge_weight, W_self0, W_neigh0, b0,
           W_self1, W_neigh1, b1):
    srcr = edge_index[0].astype(jnp.int32).reshape(NW, NCH, C)
    dstr = edge_index[1].astype(jnp.int32).reshape(NW, NCH, C)
    ewbits = lax.bitcast_convert_type(
        edge_weight.astype(_f32).reshape(NW, NCH, C), jnp.int32)
    edr = jnp.stack([srcr, dstr, ewbits], axis=2)

    z0, u0 = _tc_pre(features, W_neigh0, W_self0, b0.reshape(1, D_HID))
    zb128 = jnp.zeros((N, D_HID), _f32)
    zb16 = jnp.zeros((N, 16), _f32)
    pa, pb, dpa, dpb = _sc_agg0(z0, edr, zb128, zb16)

    z1, u1 = _tc_mid(u0, pa, pb, dpa, dpb, W_neigh1, W_self1,
                     b1.reshape(1, D_OUT))
    zb64 = jnp.zeros((N, D_OUT), _f32)
    p1a, p1b = _sc_agg1(z1, srcr, dstr, zb64)

    return _tc_post(u1, p1a, p1b, dpa, dpb)
